# Initial kernel scaffold; baseline (speedup 1.0000x reference)
import jax
import jax.numpy as jnp
from jax.experimental import pallas as pl


def kernel(features, edge_index, W1, b1, W2, b2, W3, b3):
    raise NotImplementedError("write your pallas kernel here")


# speedup vs baseline: 4.9605x; 4.9605x over previous
import functools

import jax
import jax.numpy as jnp
from jax import lax
from jax.experimental import pallas as pl
from jax.experimental.pallas import tpu as pltpu
from jax.experimental.pallas import tpu_sc as plsc

N = 10000
NP = 10240
E = 320000
D = 128

NC = 2
NS = 16
HALF = D // NC
K = 80
CH = E // (NS * K)
ROWS = NP // NS

_MESH = plsc.VectorSubcoreMesh(core_axis_name="c", subcore_axis_name="s")


@functools.partial(
    pl.kernel,
    out_type=jax.ShapeDtypeStruct((2, NP, 8), jnp.float32),
    mesh=_MESH,
    scratch_types=[
        pltpu.VMEM((CH, K), jnp.int32),
        pltpu.VMEM((K, 8), jnp.float32),
        pltpu.VMEM_SHARED((NP, 8), jnp.float32),
    ],
    compiler_params=pltpu.CompilerParams(use_tc_tiling_on_sc=False),
)
def _deg_kernel(ei_hbm, ones_hbm, zeros_hbm, out_hbm, idx_v, ones_v, count_sh):
    c = lax.axis_index("c")
    s = lax.axis_index("s")
    r0 = s * ROWS
    pltpu.sync_copy(ei_hbm.at[c, s], idx_v)
    pltpu.sync_copy(ones_hbm, ones_v)
    pltpu.sync_copy(zeros_hbm, count_sh.at[pl.ds(r0, ROWS)])
    plsc.subcore_barrier()

    def body(j, carry):
        pltpu.sync_copy(ones_v, count_sh.at[idx_v.at[j]], add=True)
        return carry

    lax.fori_loop(0, CH, body, 0)
    plsc.subcore_barrier()
    pltpu.sync_copy(count_sh.at[pl.ds(r0, ROWS)], out_hbm.at[c, pl.ds(r0, ROWS)])


@functools.partial(
    pl.kernel,
    out_type=jax.ShapeDtypeStruct((2, NP, HALF), jnp.float32),
    mesh=_MESH,
    scratch_types=[
        pltpu.VMEM((CH, K), jnp.int32),
        pltpu.VMEM((CH, K), jnp.int32),
        pltpu.VMEM((K, HALF), jnp.float32),
        pltpu.VMEM_SHARED((NP, HALF), jnp.float32),
        pltpu.SemaphoreType.DMA,
    ],
    compiler_params=pltpu.CompilerParams(use_tc_tiling_on_sc=False),
)
def _prop_kernel(t_hbm, ei_hbm, zeros_hbm, out_hbm,
                 src_v, dst_v, rows_v, agg_sh, sem):
    c = lax.axis_index("c")
    s = lax.axis_index("s")
    r0 = s * ROWS
    pltpu.sync_copy(ei_hbm.at[0, s], src_v)
    pltpu.sync_copy(ei_hbm.at[1, s], dst_v)
    pltpu.sync_copy(zeros_hbm, agg_sh.at[pl.ds(r0, ROWS)])
    plsc.subcore_barrier()

    def body(j, carry):
        pltpu.async_copy(t_hbm.at[c].at[src_v.at[j]], rows_v, sem).wait()
        pltpu.sync_copy(rows_v, agg_sh.at[dst_v.at[j]], add=True)
        return carry

    lax.fori_loop(0, CH, body, 0)
    plsc.subcore_barrier()
    pltpu.sync_copy(agg_sh.at[pl.ds(r0, ROWS)], out_hbm.at[c, pl.ds(r0, ROWS)])


def _norm(deg):
    return jnp.where(deg > 0.0, lax.rsqrt(jnp.maximum(deg, 1.0)), 0.0)


def _split_store(p, out_ref):
    out_ref[0] = p[:, :HALF]
    out_ref[1] = p[:, HALF:]


def _join(q_ref):
    return jnp.concatenate([q_ref[0], q_ref[1]], axis=1)


def _tc_first_body(feat_ref, odeg_ref, w_ref, out_ref):
    nsrc = _norm(odeg_ref[...])
    _split_store(jnp.dot(feat_ref[...] * nsrc, w_ref[...],
                         preferred_element_type=jnp.float32), out_ref)


def _tc_mid_body(q_ref, ideg_ref, odeg_ref, b_ref, w_ref, out_ref):
    ndst = _norm(ideg_ref[...])
    nsrc = _norm(odeg_ref[...])
    x = jnp.maximum(_join(q_ref) * ndst + b_ref[...], 0.0)
    _split_store(jnp.dot(x * nsrc, w_ref[...],
                         preferred_element_type=jnp.float32), out_ref)


def _tc_last_body(q_ref, ideg_ref, b_ref, out_ref):
    ndst = _norm(ideg_ref[...])
    z = _join(q_ref) * ndst + b_ref[...]
    m = jnp.max(z, axis=1, keepdims=True)
    lse = jnp.log(jnp.sum(jnp.exp(z - m), axis=1, keepdims=True)) + m
    out_ref[...] = z - lse


_SPLIT_ND = jax.ShapeDtypeStruct((2, NP, HALF), jnp.float32)
_tc_first = pl.pallas_call(_tc_first_body, out_shape=_SPLIT_ND)
_tc_mid = pl.pallas_call(_tc_mid_body, out_shape=_SPLIT_ND)
_tc_last = pl.pallas_call(_tc_last_body,
                          out_shape=jax.ShapeDtypeStruct((NP, D), jnp.float32))


def kernel(features, edge_index, W1, b1, W2, b2, W3, b3):
    ei3 = edge_index.reshape(2, NS, CH, K)
    ones8 = jnp.zeros((K, 8), jnp.float32).at[:, 0].set(1.0)
    zdeg = jnp.zeros((ROWS, 8), jnp.float32)
    zprop = jnp.zeros((ROWS, HALF), jnp.float32)
    featp = jnp.pad(features, ((0, NP - N), (0, 0)))

    degs = _deg_kernel(ei3, ones8, zdeg)
    odeg = degs[0, :, 0:1]
    ideg = degs[1, :, 0:1]

    b1r = b1.reshape(1, D)
    b2r = b2.reshape(1, D)
    b3r = b3.reshape(1, D)

    p = _tc_first(featp, odeg, W1)
    q = _prop_kernel(p, ei3, zprop)
    p = _tc_mid(q, ideg, odeg, b1r, W2)
    q = _prop_kernel(p, ei3, zprop)
    p = _tc_mid(q, ideg, odeg, b2r, W3)
    q = _prop_kernel(p, ei3, zprop)
    return _tc_last(q, ideg, b3r)[:N]


# speedup vs baseline: 6.0048x; 1.2105x over previous
import functools

import jax
import jax.numpy as jnp
from jax import lax
from jax.experimental import pallas as pl
from jax.experimental.pallas import tpu as pltpu
from jax.experimental.pallas import tpu_sc as plsc

N = 10000
NP = 10240
E = 320000
D = 128

NC = 2
NS = 16
HALF = D // NC
K = 80
CH = E // (NS * K)
ROWS = NP // NS

_MESH = plsc.VectorSubcoreMesh(core_axis_name="c", subcore_axis_name="s")


@functools.partial(
    pl.kernel,
    out_type=jax.ShapeDtypeStruct((2, NP, 8), jnp.float32),
    mesh=_MESH,
    scratch_types=[
        pltpu.VMEM((CH, K), jnp.int32),
        pltpu.VMEM((K, 8), jnp.float32),
        pltpu.VMEM_SHARED((NP, 8), jnp.float32),
    ],
    compiler_params=pltpu.CompilerParams(use_tc_tiling_on_sc=False),
)
def _deg_kernel(ei_hbm, ones_hbm, zeros_hbm, out_hbm, idx_v, ones_v, count_sh):
    c = lax.axis_index("c")
    s = lax.axis_index("s")
    r0 = s * ROWS
    pltpu.sync_copy(ei_hbm.at[c, s], idx_v)
    pltpu.sync_copy(ones_hbm, ones_v)
    pltpu.sync_copy(zeros_hbm, count_sh.at[pl.ds(r0, ROWS)])
    plsc.subcore_barrier()

    def body(j, carry):
        pltpu.sync_copy(ones_v, count_sh.at[idx_v.at[j]], add=True)
        return carry

    lax.fori_loop(0, CH, body, 0)
    plsc.subcore_barrier()
    pltpu.sync_copy(count_sh.at[pl.ds(r0, ROWS)], out_hbm.at[c, pl.ds(r0, ROWS)])


@functools.partial(
    pl.kernel,
    out_type=jax.ShapeDtypeStruct((2, NP, HALF), jnp.float32),
    mesh=_MESH,
    scratch_types=[
        pltpu.VMEM((CH, K), jnp.int32),
        pltpu.VMEM((CH, K), jnp.int32),
        pltpu.VMEM((K, HALF), jnp.float32),
        pltpu.VMEM((K, HALF), jnp.float32),
        pltpu.VMEM_SHARED((NP, HALF), jnp.float32),
        pltpu.SemaphoreType.DMA,
        pltpu.SemaphoreType.DMA,
        pltpu.SemaphoreType.DMA,
        pltpu.SemaphoreType.DMA,
    ],
    compiler_params=pltpu.CompilerParams(use_tc_tiling_on_sc=False),
)
def _prop_kernel(t_hbm, ei_hbm, zeros_hbm, out_hbm,
                 src_v, dst_v, rows0, rows1, agg_sh,
                 gsem0, gsem1, ssem0, ssem1):
    c = lax.axis_index("c")
    s = lax.axis_index("s")
    r0 = s * ROWS
    pltpu.sync_copy(ei_hbm.at[0, s], src_v)
    pltpu.sync_copy(ei_hbm.at[1, s], dst_v)
    pltpu.sync_copy(zeros_hbm, agg_sh.at[pl.ds(r0, ROWS)])
    plsc.subcore_barrier()

    rows = (rows0, rows1)
    gsem = (gsem0, gsem1)
    ssem = (ssem0, ssem1)

    def gather(j, b):
        pltpu.async_copy(t_hbm.at[c].at[src_v.at[j]], rows[b], gsem[b])

    def scatter(j, b):
        pltpu.async_copy(rows[b], agg_sh.at[dst_v.at[j]], ssem[b], add=True)

    def wait_gather(b):
        pltpu.make_async_copy(t_hbm.at[c].at[src_v.at[0]], rows[b],
                              gsem[b]).wait()

    def wait_scatter(b):
        pltpu.make_async_copy(rows[b], agg_sh.at[dst_v.at[0]],
                              ssem[b]).wait()

    gather(0, 0)

    @pl.loop(0, CH, step=2)
    def _(j0):
        wait_gather(0)
        scatter(j0, 0)

        @pl.when(j0 > 0)
        def _():
            wait_scatter(1)

        gather(j0 + 1, 1)
        wait_gather(1)
        scatter(j0 + 1, 1)
        wait_scatter(0)

        @pl.when(j0 + 2 < CH)
        def _():
            gather(j0 + 2, 0)

    wait_scatter(1)
    plsc.subcore_barrier()
    pltpu.sync_copy(agg_sh.at[pl.ds(r0, ROWS)], out_hbm.at[c, pl.ds(r0, ROWS)])


def _norm(deg):
    return jnp.where(deg > 0.0, lax.rsqrt(jnp.maximum(deg, 1.0)), 0.0)


def _split_store(p, out_ref):
    out_ref[0] = p[:, :HALF]
    out_ref[1] = p[:, HALF:]


def _join(q_ref):
    return jnp.concatenate([q_ref[0], q_ref[1]], axis=1)


def _tc_first_body(feat_ref, odeg_ref, w_ref, out_ref):
    nsrc = _norm(odeg_ref[...])
    _split_store(jnp.dot(feat_ref[...] * nsrc, w_ref[...],
                         preferred_element_type=jnp.float32), out_ref)


def _tc_mid_body(q_ref, ideg_ref, odeg_ref, b_ref, w_ref, out_ref):
    ndst = _norm(ideg_ref[...])
    nsrc = _norm(odeg_ref[...])
    x = jnp.maximum(_join(q_ref) * ndst + b_ref[...], 0.0)
    _split_store(jnp.dot(x * nsrc, w_ref[...],
                         preferred_element_type=jnp.float32), out_ref)


def _tc_last_body(q_ref, ideg_ref, b_ref, out_ref):
    ndst = _norm(ideg_ref[...])
    z = _join(q_ref) * ndst + b_ref[...]
    m = jnp.max(z, axis=1, keepdims=True)
    lse = jnp.log(jnp.sum(jnp.exp(z - m), axis=1, keepdims=True)) + m
    out_ref[...] = z - lse


_SPLIT_ND = jax.ShapeDtypeStruct((2, NP, HALF), jnp.float32)
_tc_first = pl.pallas_call(_tc_first_body, out_shape=_SPLIT_ND)
_tc_mid = pl.pallas_call(_tc_mid_body, out_shape=_SPLIT_ND)
_tc_last = pl.pallas_call(_tc_last_body,
                          out_shape=jax.ShapeDtypeStruct((NP, D), jnp.float32))


def kernel(features, edge_index, W1, b1, W2, b2, W3, b3):
    ei3 = edge_index.reshape(2, NS, CH, K)
    ones8 = jnp.zeros((K, 8), jnp.float32).at[:, 0].set(1.0)
    zdeg = jnp.zeros((ROWS, 8), jnp.float32)
    zprop = jnp.zeros((ROWS, HALF), jnp.float32)
    featp = jnp.pad(features, ((0, NP - N), (0, 0)))

    degs = _deg_kernel(ei3, ones8, zdeg)
    odeg = degs[0, :, 0:1]
    ideg = degs[1, :, 0:1]

    b1r = b1.reshape(1, D)
    b2r = b2.reshape(1, D)
    b3r = b3.reshape(1, D)

    p = _tc_first(featp, odeg, W1)
    q = _prop_kernel(p, ei3, zprop)
    p = _tc_mid(q, ideg, odeg, b1r, W2)
    q = _prop_kernel(p, ei3, zprop)
    p = _tc_mid(q, ideg, odeg, b2r, W3)
    q = _prop_kernel(p, ei3, zprop)
    return _tc_last(q, ideg, b3r)[:N]


# speedup vs baseline: 6.7208x; 1.1192x over previous
import functools

import jax
import jax.numpy as jnp
from jax import lax
from jax.experimental import pallas as pl
from jax.experimental.pallas import tpu as pltpu
from jax.experimental.pallas import tpu_sc as plsc

N = 10000
NP = 10240
E = 320000
D = 128

NC = 2
NS = 16
HALF = D // NC
K = 80
CH = E // (NS * K)
ROWS = NP // NS

_MESH = plsc.VectorSubcoreMesh(core_axis_name="c", subcore_axis_name="s")


@functools.partial(
    pl.kernel,
    out_type=jax.ShapeDtypeStruct((2, NP, 8), jnp.float32),
    mesh=_MESH,
    scratch_types=[
        pltpu.VMEM((CH, K), jnp.int32),
        pltpu.VMEM((K, 8), jnp.float32),
        pltpu.VMEM_SHARED((NP, 8), jnp.float32),
    ],
    compiler_params=pltpu.CompilerParams(use_tc_tiling_on_sc=False),
)
def _deg_kernel(ei_hbm, ones_hbm, zeros_hbm, out_hbm, idx_v, ones_v, count_sh):
    c = lax.axis_index("c")
    s = lax.axis_index("s")
    r0 = s * ROWS
    pltpu.sync_copy(ei_hbm.at[c, s], idx_v)
    pltpu.sync_copy(ones_hbm, ones_v)
    pltpu.sync_copy(zeros_hbm, count_sh.at[pl.ds(r0, ROWS)])
    plsc.subcore_barrier()

    def body(j, carry):
        pltpu.sync_copy(ones_v, count_sh.at[idx_v.at[j]], add=True)
        return carry

    lax.fori_loop(0, CH, body, 0)
    plsc.subcore_barrier()
    pltpu.sync_copy(count_sh.at[pl.ds(r0, ROWS)], out_hbm.at[c, pl.ds(r0, ROWS)])


@functools.partial(
    pl.kernel,
    out_type=jax.ShapeDtypeStruct((2, NP, HALF), jnp.bfloat16),
    mesh=_MESH,
    scratch_types=[
        pltpu.VMEM((CH, K), jnp.int32),
        pltpu.VMEM((CH, K), jnp.int32),
        pltpu.VMEM((K, HALF), jnp.bfloat16),
        pltpu.VMEM((K, HALF), jnp.bfloat16),
        pltpu.VMEM_SHARED((NP, HALF), jnp.bfloat16),
        pltpu.SemaphoreType.DMA,
        pltpu.SemaphoreType.DMA,
        pltpu.SemaphoreType.DMA,
        pltpu.SemaphoreType.DMA,
    ],
    compiler_params=pltpu.CompilerParams(use_tc_tiling_on_sc=False),
)
def _prop_kernel(t_hbm, ei_hbm, zeros_hbm, out_hbm,
                 src_v, dst_v, rows0, rows1, agg_sh,
                 gsem0, gsem1, ssem0, ssem1):
    c = lax.axis_index("c")
    s = lax.axis_index("s")
    r0 = s * ROWS
    pltpu.sync_copy(ei_hbm.at[0, s], src_v)
    pltpu.sync_copy(ei_hbm.at[1, s], dst_v)
    pltpu.sync_copy(zeros_hbm, agg_sh.at[pl.ds(r0, ROWS)])
    plsc.subcore_barrier()

    rows = (rows0, rows1)
    gsem = (gsem0, gsem1)
    ssem = (ssem0, ssem1)

    def gather(j, b):
        pltpu.async_copy(t_hbm.at[c].at[src_v.at[j]], rows[b], gsem[b])

    def scatter(j, b):
        pltpu.async_copy(rows[b], agg_sh.at[dst_v.at[j]], ssem[b], add=True)

    def wait_gather(b):
        pltpu.make_async_copy(t_hbm.at[c].at[src_v.at[0]], rows[b],
                              gsem[b]).wait()

    def wait_scatter(b):
        pltpu.make_async_copy(rows[b], agg_sh.at[dst_v.at[0]],
                              ssem[b]).wait()

    gather(0, 0)

    @pl.loop(0, CH, step=2)
    def _(j0):
        wait_gather(0)
        scatter(j0, 0)

        @pl.when(j0 > 0)
        def _():
            wait_scatter(1)

        gather(j0 + 1, 1)
        wait_gather(1)
        scatter(j0 + 1, 1)
        wait_scatter(0)

        @pl.when(j0 + 2 < CH)
        def _():
            gather(j0 + 2, 0)

    wait_scatter(1)
    plsc.subcore_barrier()
    pltpu.sync_copy(agg_sh.at[pl.ds(r0, ROWS)], out_hbm.at[c, pl.ds(r0, ROWS)])


def _norm(deg):
    return jnp.where(deg > 0.0, lax.rsqrt(jnp.maximum(deg, 1.0)), 0.0)


def _split_store(p, out_ref):
    pb = p.astype(jnp.bfloat16)
    out_ref[0] = pb[:, :HALF]
    out_ref[1] = pb[:, HALF:]


def _join(q_ref):
    return jnp.concatenate([q_ref[0], q_ref[1]], axis=1).astype(jnp.float32)


def _tc_first_body(feat_ref, odeg_ref, w_ref, out_ref):
    nsrc = _norm(odeg_ref[...])
    _split_store(jnp.dot(feat_ref[...] * nsrc, w_ref[...],
                         preferred_element_type=jnp.float32), out_ref)


def _tc_mid_body(q_ref, ideg_ref, odeg_ref, b_ref, w_ref, out_ref):
    ndst = _norm(ideg_ref[...])
    nsrc = _norm(odeg_ref[...])
    x = jnp.maximum(_join(q_ref) * ndst + b_ref[...], 0.0)
    _split_store(jnp.dot(x * nsrc, w_ref[...],
                         preferred_element_type=jnp.float32), out_ref)


def _tc_last_body(q_ref, ideg_ref, b_ref, out_ref):
    ndst = _norm(ideg_ref[...])
    z = _join(q_ref) * ndst + b_ref[...]
    m = jnp.max(z, axis=1, keepdims=True)
    lse = jnp.log(jnp.sum(jnp.exp(z - m), axis=1, keepdims=True)) + m
    out_ref[...] = z - lse


_SPLIT_ND = jax.ShapeDtypeStruct((2, NP, HALF), jnp.bfloat16)
_tc_first = pl.pallas_call(_tc_first_body, out_shape=_SPLIT_ND)
_tc_mid = pl.pallas_call(_tc_mid_body, out_shape=_SPLIT_ND)
_tc_last = pl.pallas_call(_tc_last_body,
                          out_shape=jax.ShapeDtypeStruct((NP, D), jnp.float32))


def kernel(features, edge_index, W1, b1, W2, b2, W3, b3):
    ei3 = edge_index.reshape(2, NS, CH, K)
    ones8 = jnp.zeros((K, 8), jnp.float32).at[:, 0].set(1.0)
    zdeg = jnp.zeros((ROWS, 8), jnp.float32)
    zprop = jnp.zeros((ROWS, HALF), jnp.bfloat16)
    featp = jnp.pad(features, ((0, NP - N), (0, 0)))

    degs = _deg_kernel(ei3, ones8, zdeg)
    odeg = degs[0, :, 0:1]
    ideg = degs[1, :, 0:1]

    b1r = b1.reshape(1, D)
    b2r = b2.reshape(1, D)
    b3r = b3.reshape(1, D)

    p = _tc_first(featp, odeg, W1)
    q = _prop_kernel(p, ei3, zprop)
    p = _tc_mid(q, ideg, odeg, b1r, W2)
    q = _prop_kernel(p, ei3, zprop)
    p = _tc_mid(q, ideg, odeg, b2r, W3)
    q = _prop_kernel(p, ei3, zprop)
    return _tc_last(q, ideg, b3r)[:N]


# speedup vs baseline: 7.3212x; 1.0893x over previous
import functools

import jax
import jax.numpy as jnp
from jax import lax
from jax.experimental import pallas as pl
from jax.experimental.pallas import tpu as pltpu
from jax.experimental.pallas import tpu_sc as plsc

N = 10000
NP = 10240
E = 320000
D = 128

NC = 2
NS = 16
K = 80
CHD = E // (NS * K)
CHP = 126
EP = NC * NS * CHP * K
ROWS = NP // NS

_MESH = plsc.VectorSubcoreMesh(core_axis_name="c", subcore_axis_name="s")


@functools.partial(
    pl.kernel,
    out_type=jax.ShapeDtypeStruct((2, NP, 8), jnp.float32),
    mesh=_MESH,
    scratch_types=[
        pltpu.VMEM((CHD, K), jnp.int32),
        pltpu.VMEM((K, 8), jnp.float32),
        pltpu.VMEM_SHARED((NP, 8), jnp.float32),
    ],
    compiler_params=pltpu.CompilerParams(use_tc_tiling_on_sc=False),
)
def _deg_kernel(ei_hbm, ones_hbm, zeros_hbm, out_hbm, idx_v, ones_v, count_sh):
    c = lax.axis_index("c")
    s = lax.axis_index("s")
    r0 = s * ROWS
    pltpu.sync_copy(ei_hbm.at[c, s], idx_v)
    pltpu.sync_copy(ones_hbm, ones_v)
    pltpu.sync_copy(zeros_hbm, count_sh.at[pl.ds(r0, ROWS)])
    plsc.subcore_barrier()

    def body(j, carry):
        pltpu.sync_copy(ones_v, count_sh.at[idx_v.at[j]], add=True)
        return carry

    lax.fori_loop(0, CHD, body, 0)
    plsc.subcore_barrier()
    pltpu.sync_copy(count_sh.at[pl.ds(r0, ROWS)], out_hbm.at[c, pl.ds(r0, ROWS)])


@functools.partial(
    pl.kernel,
    out_type=jax.ShapeDtypeStruct((2, NP, D), jnp.bfloat16),
    mesh=_MESH,
    scratch_types=[
        pltpu.VMEM((CHP, K), jnp.int32),
        pltpu.VMEM((CHP, K), jnp.int32),
        pltpu.VMEM((K, D), jnp.bfloat16),
        pltpu.VMEM((K, D), jnp.bfloat16),
        pltpu.VMEM_SHARED((NP, D), jnp.bfloat16),
        pltpu.SemaphoreType.DMA,
        pltpu.SemaphoreType.DMA,
        pltpu.SemaphoreType.DMA,
        pltpu.SemaphoreType.DMA,
    ],
    compiler_params=pltpu.CompilerParams(use_tc_tiling_on_sc=False),
)
def _prop_kernel(t_hbm, ei_hbm, zeros_hbm, out_hbm,
                 src_v, dst_v, rows0, rows1, agg_sh,
                 gsem0, gsem1, ssem0, ssem1):
    c = lax.axis_index("c")
    s = lax.axis_index("s")
    r0 = s * ROWS
    pltpu.sync_copy(ei_hbm.at[0, c, s], src_v)
    pltpu.sync_copy(ei_hbm.at[1, c, s], dst_v)
    pltpu.sync_copy(zeros_hbm, agg_sh.at[pl.ds(r0, ROWS)])
    plsc.subcore_barrier()

    rows = (rows0, rows1)
    gsem = (gsem0, gsem1)
    ssem = (ssem0, ssem1)

    def gather(j, b):
        pltpu.async_copy(t_hbm.at[src_v.at[j]], rows[b], gsem[b])

    def scatter(j, b):
        pltpu.async_copy(rows[b], agg_sh.at[dst_v.at[j]], ssem[b], add=True)

    def wait_gather(b):
        pltpu.make_async_copy(t_hbm.at[src_v.at[0]], rows[b], gsem[b]).wait()

    def wait_scatter(b):
        pltpu.make_async_copy(rows[b], agg_sh.at[dst_v.at[0]], ssem[b]).wait()

    gather(0, 0)

    @pl.loop(0, CHP, step=2)
    def _(j0):
        wait_gather(0)
        scatter(j0, 0)

        @pl.when(j0 > 0)
        def _():
            wait_scatter(1)

        gather(j0 + 1, 1)
        wait_gather(1)
        scatter(j0 + 1, 1)
        wait_scatter(0)

        @pl.when(j0 + 2 < CHP)
        def _():
            gather(j0 + 2, 0)

    wait_scatter(1)
    plsc.subcore_barrier()
    pltpu.sync_copy(agg_sh.at[pl.ds(r0, ROWS)], out_hbm.at[c, pl.ds(r0, ROWS)])


def _norm(deg):
    return jnp.where(deg > 0.0, lax.rsqrt(jnp.maximum(deg, 1.0)), 0.0)


def _join(q_ref):
    return q_ref[0].astype(jnp.float32) + q_ref[1].astype(jnp.float32)


def _tc_first_body(feat_ref, odeg_ref, w_ref, out_ref):
    nsrc = _norm(odeg_ref[...])
    p = jnp.dot(feat_ref[...] * nsrc, w_ref[...],
                preferred_element_type=jnp.float32)
    out_ref[...] = p.astype(jnp.bfloat16)


def _tc_mid_body(q_ref, ideg_ref, odeg_ref, b_ref, w_ref, out_ref):
    ndst = _norm(ideg_ref[...])
    nsrc = _norm(odeg_ref[...])
    x = jnp.maximum(_join(q_ref) * ndst + b_ref[...], 0.0)
    p = jnp.dot(x * nsrc, w_ref[...], preferred_element_type=jnp.float32)
    out_ref[...] = p.astype(jnp.bfloat16)


def _tc_last_body(q_ref, ideg_ref, b_ref, out_ref):
    ndst = _norm(ideg_ref[...])
    z = _join(q_ref) * ndst + b_ref[...]
    m = jnp.max(z, axis=1, keepdims=True)
    lse = jnp.log(jnp.sum(jnp.exp(z - m), axis=1, keepdims=True)) + m
    out_ref[...] = z - lse


_T_ND = jax.ShapeDtypeStruct((NP, D), jnp.bfloat16)
_tc_first = pl.pallas_call(_tc_first_body, out_shape=_T_ND)
_tc_mid = pl.pallas_call(_tc_mid_body, out_shape=_T_ND)
_tc_last = pl.pallas_call(_tc_last_body,
                          out_shape=jax.ShapeDtypeStruct((NP, D), jnp.float32))


def kernel(features, edge_index, W1, b1, W2, b2, W3, b3):
    ei3 = edge_index.reshape(2, NS, CHD, K)
    eip = jnp.pad(edge_index, ((0, 0), (0, EP - E)), constant_values=N)
    ei4 = eip.reshape(2, NC, NS, CHP, K)

    ones8 = jnp.zeros((K, 8), jnp.float32).at[:, 0].set(1.0)
    zdeg = jnp.zeros((ROWS, 8), jnp.float32)
    zprop = jnp.zeros((ROWS, D), jnp.bfloat16)
    featp = jnp.pad(features, ((0, NP - N), (0, 0)))

    degs = _deg_kernel(ei3, ones8, zdeg)
    odeg = degs[0, :, 0:1]
    ideg = degs[1, :, 0:1]

    b1r = b1.reshape(1, D)
    b2r = b2.reshape(1, D)
    b3r = b3.reshape(1, D)

    p = _tc_first(featp, odeg, W1)
    q = _prop_kernel(p, ei4, zprop)
    p = _tc_mid(q, ideg, odeg, b1r, W2)
    q = _prop_kernel(p, ei4, zprop)
    p = _tc_mid(q, ideg, odeg, b2r, W3)
    q = _prop_kernel(p, ei4, zprop)
    return _tc_last(q, ideg, b3r)[:N]


# speedup vs baseline: 8.9927x; 1.2283x over previous
import functools

import jax
import jax.numpy as jnp
from jax import lax
from jax.experimental import pallas as pl
from jax.experimental.pallas import tpu as pltpu
from jax.experimental.pallas import tpu_sc as plsc

N = 10000
NP = 10240
E = 320000
D = 128

NC = 2
NS = 16
K = 80
CHD = E // (NS * K)
CHP = 126
EP = NC * NS * CHP * K
ROWS = NP // NS

_MESH = plsc.VectorSubcoreMesh(core_axis_name="c", subcore_axis_name="s")


@functools.partial(
    pl.kernel,
    out_type=jax.ShapeDtypeStruct((2, NP, 8), jnp.float32),
    mesh=_MESH,
    scratch_types=[
        pltpu.VMEM((CHD, K), jnp.int32),
        pltpu.VMEM((K, 8), jnp.float32),
        pltpu.VMEM_SHARED((NP, 8), jnp.float32),
    ],
    compiler_params=pltpu.CompilerParams(use_tc_tiling_on_sc=False),
)
def _deg_kernel(ei_hbm, ones_hbm, zeros_hbm, out_hbm, idx_v, ones_v, count_sh):
    c = lax.axis_index("c")
    s = lax.axis_index("s")
    r0 = s * ROWS
    pltpu.sync_copy(ei_hbm.at[c, s], idx_v)
    pltpu.sync_copy(ones_hbm, ones_v)
    pltpu.sync_copy(zeros_hbm, count_sh.at[pl.ds(r0, ROWS)])
    plsc.subcore_barrier()

    def body(j, carry):
        pltpu.sync_copy(ones_v, count_sh.at[idx_v.at[j]], add=True)
        return carry

    lax.fori_loop(0, CHD, body, 0)
    plsc.subcore_barrier()
    pltpu.sync_copy(count_sh.at[pl.ds(r0, ROWS)], out_hbm.at[c, pl.ds(r0, ROWS)])


@functools.partial(
    pl.kernel,
    out_type=jax.ShapeDtypeStruct((2, NP, D), jnp.bfloat16),
    mesh=_MESH,
    scratch_types=[
        pltpu.VMEM((CHP, K), jnp.int32),
        pltpu.VMEM((CHP, K), jnp.int32),
        pltpu.VMEM((K, D), jnp.bfloat16),
        pltpu.VMEM((K, D), jnp.bfloat16),
        pltpu.VMEM_SHARED((NP, D), jnp.bfloat16),
        pltpu.SemaphoreType.DMA,
        pltpu.SemaphoreType.DMA,
        pltpu.SemaphoreType.DMA,
        pltpu.SemaphoreType.DMA,
    ],
    compiler_params=pltpu.CompilerParams(use_tc_tiling_on_sc=False),
)
def _prop_kernel(t_hbm, ei_hbm, zeros_hbm, out_hbm,
                 src_v, dst_v, rows0, rows1, agg_sh,
                 gsem0, gsem1, ssem0, ssem1):
    c = lax.axis_index("c")
    s = lax.axis_index("s")
    r0 = s * ROWS
    pltpu.sync_copy(ei_hbm.at[0, c, s], src_v)
    pltpu.sync_copy(ei_hbm.at[1, c, s], dst_v)
    pltpu.sync_copy(zeros_hbm, agg_sh.at[pl.ds(r0, ROWS)])
    plsc.subcore_barrier()

    rows = (rows0, rows1)
    gsem = (gsem0, gsem1)
    ssem = (ssem0, ssem1)

    def gather(j, b):
        pltpu.async_copy(t_hbm.at[src_v.at[j]], rows[b], gsem[b])

    def scatter(j, b):
        pltpu.async_copy(rows[b], agg_sh.at[dst_v.at[j]], ssem[b], add=True)

    def wait_gather(b):
        pltpu.make_async_copy(t_hbm.at[src_v.at[0]], rows[b], gsem[b]).wait()

    def wait_scatter(b):
        pltpu.make_async_copy(rows[b], agg_sh.at[dst_v.at[0]], ssem[b]).wait()

    gather(0, 0)

    @pl.loop(0, CHP, step=2)
    def _(j0):
        wait_gather(0)
        scatter(j0, 0)

        @pl.when(j0 > 0)
        def _():
            wait_scatter(1)

        gather(j0 + 1, 1)
        wait_gather(1)
        scatter(j0 + 1, 1)
        wait_scatter(0)

        @pl.when(j0 + 2 < CHP)
        def _():
            gather(j0 + 2, 0)

    wait_scatter(1)
    plsc.subcore_barrier()
    pltpu.sync_copy(agg_sh.at[pl.ds(r0, ROWS)], out_hbm.at[c, pl.ds(r0, ROWS)])


def _norm(deg):
    return jnp.where(deg > 0.0, lax.rsqrt(jnp.maximum(deg, 1.0)), 0.0)


def _join(q_ref):
    return q_ref[0].astype(jnp.float32) + q_ref[1].astype(jnp.float32)


def _tc_first_body(feat_ref, odeg_ref, w_ref, out_ref):
    nsrc = _norm(odeg_ref[...])
    p = jnp.dot(feat_ref[...] * nsrc, w_ref[...],
                preferred_element_type=jnp.float32)
    out_ref[...] = p.astype(jnp.bfloat16)


def _tc_mid_body(q_ref, ideg_ref, odeg_ref, b_ref, w_ref, out_ref):
    ndst = _norm(ideg_ref[...])
    nsrc = _norm(odeg_ref[...])
    x = jnp.maximum(_join(q_ref) * ndst + b_ref[...], 0.0)
    p = jnp.dot(x * nsrc, w_ref[...], preferred_element_type=jnp.float32)
    out_ref[...] = p.astype(jnp.bfloat16)


def _tc_last_body(q_ref, ideg_ref, b_ref, out_ref):
    ndst = _norm(ideg_ref[...])
    z = _join(q_ref) * ndst + b_ref[...]
    m = jnp.max(z, axis=1, keepdims=True)
    lse = jnp.log(jnp.sum(jnp.exp(z - m), axis=1, keepdims=True)) + m
    out_ref[...] = z - lse


_T_ND = jax.ShapeDtypeStruct((NP, D), jnp.bfloat16)
_tc_first = pl.pallas_call(_tc_first_body, out_shape=_T_ND)
_tc_mid = pl.pallas_call(_tc_mid_body, out_shape=_T_ND)
_tc_last = pl.pallas_call(_tc_last_body,
                          out_shape=jax.ShapeDtypeStruct((NP, D), jnp.float32))


def kernel(features, edge_index, W1, b1, W2, b2, W3, b3):
    ei3 = edge_index.reshape(2, NS, CHD, K)
    padrow = (N + jnp.arange(EP - E, dtype=jnp.int32) % (NP - N))
    eip = jnp.concatenate(
        [edge_index, jnp.stack([padrow, padrow])], axis=1)
    ei4 = eip.reshape(2, NC, NS, CHP, K)

    ones8 = jnp.zeros((K, 8), jnp.float32).at[:, 0].set(1.0)
    zdeg = jnp.zeros((ROWS, 8), jnp.float32)
    zprop = jnp.zeros((ROWS, D), jnp.bfloat16)
    featp = jnp.pad(features, ((0, NP - N), (0, 0)))

    degs = _deg_kernel(ei3, ones8, zdeg)
    odeg = degs[0, :, 0:1]
    ideg = degs[1, :, 0:1]

    b1r = b1.reshape(1, D)
    b2r = b2.reshape(1, D)
    b3r = b3.reshape(1, D)

    p = _tc_first(featp, odeg, W1)
    q = _prop_kernel(p, ei4, zprop)
    p = _tc_mid(q, ideg, odeg, b1r, W2)
    q = _prop_kernel(p, ei4, zprop)
    p = _tc_mid(q, ideg, odeg, b2r, W3)
    q = _prop_kernel(p, ei4, zprop)
    return _tc_last(q, ideg, b3r)[:N]
